# baseline (speedup 1.0000x reference)
import functools

import jax
import jax.numpy as jnp
from jax import lax
from jax.experimental import pallas as pl
from jax.experimental.pallas import tpu as pltpu
from jax.experimental.pallas import tpu_sc as plsc

N = 10000
E = 320000
D = 128
DE = 16
H = 4
F = 64
HF = H * F
HHALF = HF // 2

NSUB = 16
EPER = E // NSUB
K = 80
SK = 2000
NSK = EPER // SK
NC2 = SK // K
NQ = NC2 // 4
NPT = 624
NTAIL = N - NSUB * NPT

NW = 32
EW = E // NW


def _matT(x, w):
    return lax.dot_general(x, w, (((1,), (1,)), ((), ())),
                           preferred_element_type=jnp.float32)


def _proj_body(x_ref, ws_ref, wc_ref, wd_ref, bd_ref, fe_ref, we_ref,
               fcs_ref, attn_ref, fdst_ref, aeT_ref):
    x = x_ref[...]
    fcs_ref[...] = _matT(x, ws_ref[...])
    attn_ref[...] = _matT(x, wc_ref[...])
    fdst_ref[...] = _matT(x, wd_ref[...]) + bd_ref[...]
    aeT_ref[...] = lax.dot_general(
        we_ref[...], fe_ref[...], (((1,), (1,)), ((), ())),
        preferred_element_type=jnp.float32)


def _node_proj(feat_src, W_src, W_cat, W_dst, b_dst, feat_edge, W_attn_edge):
    R = 1000
    GN = N // R
    EB = E // (2 * GN)
    return pl.pallas_call(
        _proj_body,
        grid=(2, GN),
        in_specs=[
            pl.BlockSpec((R, D), lambda h, i: (i, 0)),
            pl.BlockSpec((HHALF, D), lambda h, i: (h, 0)),
            pl.BlockSpec((2 * H, D), lambda h, i: (0, 0)),
            pl.BlockSpec((HF, D), lambda h, i: (0, 0)),
            pl.BlockSpec((1, HF), lambda h, i: (0, 0)),
            pl.BlockSpec((EB, DE), lambda h, i: (h * GN + i, 0)),
            pl.BlockSpec((H, DE), lambda h, i: (0, 0)),
        ],
        out_specs=[
            pl.BlockSpec((R, HHALF), lambda h, i: (h * GN + i, 0)),
            pl.BlockSpec((R, 2 * H), lambda h, i: (i, 0)),
            pl.BlockSpec((R, HF), lambda h, i: (i, 0)),
            pl.BlockSpec((H, EB), lambda h, i: (0, h * GN + i)),
        ],
        out_shape=[
            jax.ShapeDtypeStruct((2 * N, HHALF), jnp.float32),
            jax.ShapeDtypeStruct((N, 2 * H), jnp.float32),
            jax.ShapeDtypeStruct((N, HF), jnp.float32),
            jax.ShapeDtypeStruct((H, E), jnp.float32),
        ],
    )(feat_src, W_src, W_cat, W_dst, b_dst, feat_edge, W_attn_edge)


def _sc_attn_kernel(attn_flat, aeT, src, dst):
    mesh = plsc.VectorSubcoreMesh(core_axis_name="c", subcore_axis_name="s")

    @functools.partial(
        pl.kernel,
        out_type=jax.ShapeDtypeStruct((H * E,), jnp.float32),
        mesh=mesh,
        scratch_types=[
            pltpu.VMEM((N * 2 * H,), jnp.float32),
            pltpu.VMEM((EW,), jnp.int32),
            pltpu.VMEM((EW,), jnp.int32),
            [pltpu.VMEM((EW,), jnp.float32) for _ in range(2)],
        ],
        compiler_params=pltpu.CompilerParams(needs_layout_passes=False),
    )
    def k(attn_hbm, aeT_hbm, src_hbm, dst_hbm, a_hbm,
          tbl, srcv, dstv, aebs):
        c = lax.axis_index("c")
        s = lax.axis_index("s")
        w = s * 2 + c
        base = w * EW
        pltpu.sync_copy(attn_hbm, tbl)
        pltpu.sync_copy(src_hbm.at[pl.ds(base, EW)], srcv)
        pltpu.sync_copy(dst_hbm.at[pl.ds(base, EW)], dstv)

        for hp in range(2):
            for hl in range(2):
                h = 2 * hp + hl
                pltpu.sync_copy(aeT_hbm.at[pl.ds(h * E + base, EW)],
                                aebs[hl])

            def group(g, carry2):
                off = g * 16
                sidx = srcv[pl.ds(off, 16)] * (2 * H)
                didx = dstv[pl.ds(off, 16)] * (2 * H)
                for hl in range(2):
                    h = 2 * hp + hl
                    asrc = plsc.load_gather(tbl, [sidx + h])
                    adst = plsc.load_gather(tbl, [didx + (H + h)])
                    e = asrc + adst + aebs[hl][pl.ds(off, 16)]
                    aebs[hl][pl.ds(off, 16)] = jnp.where(e > 0, e, e * 0.2)
                return carry2

            lax.fori_loop(0, EW // 16, group, 0)
            for hl in range(2):
                h = 2 * hp + hl
                pltpu.sync_copy(aebs[hl], a_hbm.at[pl.ds(h * E + base, EW)])

    return k(attn_flat, aeT, src, dst)


def _sc_edge_kernel(fcS, a_hbm_in, src, dst4d, zeros_hbm):
    mesh = plsc.VectorSubcoreMesh(core_axis_name="c", subcore_axis_name="s")

    @functools.partial(
        pl.kernel,
        out_type=jax.ShapeDtypeStruct((2 * N, HHALF), jnp.float32),
        mesh=mesh,
        scratch_types=[
            [pltpu.VMEM((K, HHALF), jnp.float32) for _ in range(4)],
            pltpu.VMEM((SK,), jnp.int32),
            pltpu.VMEM((NC2, K), jnp.int32),
            [pltpu.VMEM((K,), jnp.float32) for _ in range(4)],
            [pltpu.VMEM((K,), jnp.float32) for _ in range(4)],
            pltpu.VMEM_SHARED((N, HHALF), jnp.float32),
            [pltpu.SemaphoreType.DMA for _ in range(4)],
            [pltpu.SemaphoreType.DMA for _ in range(4)],
            pltpu.SemaphoreType.DMA,
        ],
        compiler_params=pltpu.CompilerParams(needs_layout_passes=False),
    )
    def k(fcs_hbm, a_hbm, src_hbm, dst4d_hbm, z_hbm, agg_hbm,
          rows, srcv, dstv2d, ablo, abhi, accum, gsems, ssems, isem):
        c = lax.axis_index("c")
        s = lax.axis_index("s")
        h0 = 2 * c
        cN = c * N
        rbase = s * NPT

        pltpu.sync_copy(z_hbm, accum.at[pl.ds(rbase, NPT)])

        @pl.when(s == 0)
        def _():
            pltpu.sync_copy(z_hbm.at[pl.ds(0, NTAIL)],
                            accum.at[pl.ds(NSUB * NPT, NTAIL)])

        plsc.subcore_barrier()

        def compute(b, cd):
            def group(g, carry):
                off = g * 16
                a0 = ablo[b][pl.ds(off, 16)]
                a1 = abhi[b][pl.ds(off, 16)]
                for j in range(16):
                    bidx = jnp.full((16,), j, jnp.int32)
                    erow = off + j
                    for hl, av in ((0, a0), (1, a1)):
                        bc = av.at[bidx].get(mode="promise_in_bounds")
                        for q in range(F // 16):
                            cs = hl * F + q * 16
                            rows[b][erow, pl.ds(cs, 16)] = (
                                rows[b][erow, pl.ds(cs, 16)] * bc)
                return carry

            lax.fori_loop(0, K // 16, group, 0)

        def g_start(b, cd, base):
            pltpu.async_copy(fcs_hbm.at[srcv.at[pl.ds(cd * K, K)]],
                             rows[b], gsems[b])
            ebase = base + cd * K
            pltpu.async_copy(a_hbm.at[pl.ds(h0 * E + ebase, K)], ablo[b],
                             gsems[b])
            pltpu.async_copy(a_hbm.at[pl.ds((h0 + 1) * E + ebase, K)],
                             abhi[b], gsems[b])

        def g_wait(b, cd, base):
            pltpu.make_async_copy(fcs_hbm.at[srcv.at[pl.ds(cd * K, K)]],
                                  rows[b], gsems[b]).wait()
            ebase = base + cd * K
            pltpu.make_async_copy(a_hbm.at[pl.ds(h0 * E + ebase, K)],
                                  ablo[b], gsems[b]).wait()
            pltpu.make_async_copy(a_hbm.at[pl.ds((h0 + 1) * E + ebase, K)],
                                  abhi[b], gsems[b]).wait()

        def sc_start(b, cd):
            pltpu.async_copy(rows[b], accum.at[dstv2d.at[cd]], ssems[b],
                             add=True)

        def sc_wait(b, cd):
            pltpu.make_async_copy(rows[b], accum.at[dstv2d.at[cd]],
                                  ssems[b]).wait()

        def sk_body(j, carry):
            base = s * EPER + j * SK
            pltpu.async_copy(src_hbm.at[pl.ds(base, SK)], srcv, isem)
            pltpu.async_copy(dst4d_hbm.at[s, j], dstv2d, isem)
            pltpu.make_async_copy(src_hbm.at[pl.ds(base, SK)], srcv,
                                  isem).wait()
            pltpu.make_async_copy(dst4d_hbm.at[s, j], dstv2d, isem).wait()

            def adj(g, carry2):
                srcv[pl.ds(g * 16, 16)] = srcv[pl.ds(g * 16, 16)] + cN
                return carry2

            lax.fori_loop(0, SK // 16, adj, 0)
            g_start(0, 0, base)
            g_start(1, 1, base)

            def quad(p, carry2):
                c0 = 4 * p
                for t in range(4):
                    cd = c0 + t
                    b = t
                    b2 = (t + 2) % 4
                    if t >= 2:
                        sc_wait(b2, cd - 2)
                    else:
                        @pl.when(p > 0)
                        def _():
                            sc_wait(b2, cd - 2)
                    if t < 3:
                        g_start(b2, cd + 2, base)
                    else:
                        @pl.when(p < NQ - 1)
                        def _():
                            g_start(b2, cd + 2, base)
                    g_wait(b, cd, base)
                    compute(b, cd)
                    sc_start(b, cd)
                return carry2

            lax.fori_loop(0, NQ, quad, 0)
            sc_wait(2, NC2 - 3)
            g_wait(0, NC2 - 1, base)
            compute(0, NC2 - 1)
            sc_start(0, NC2 - 1)
            sc_wait(3, NC2 - 2)
            sc_wait(0, NC2 - 1)
            return carry

        lax.fori_loop(0, NSK, sk_body, 0)
        plsc.subcore_barrier()
        pltpu.sync_copy(accum.at[pl.ds(rbase, NPT)],
                        agg_hbm.at[pl.ds(cN + rbase, NPT)])

        @pl.when(s == 0)
        def _():
            pltpu.sync_copy(accum.at[pl.ds(NSUB * NPT, NTAIL)],
                            agg_hbm.at[pl.ds(cN + NSUB * NPT, NTAIL)])

    return k(fcS, a_hbm_in, src, dst4d, zeros_hbm)


def _final_body(a0_ref, a1_ref, fdst_ref, wa_ref, ba_ref, sc_ref, of_ref,
                out_ref):
    x = jnp.concatenate([a0_ref[...], a1_ref[...]], axis=1)
    r = lax.broadcasted_iota(jnp.int32, (HF, HF), 0) // F
    cc = lax.broadcasted_iota(jnp.int32, (HF, HF), 1) // F
    M = jnp.where(r == cc, jnp.float32(1.0 / F), jnp.float32(0.0))
    mean = jnp.dot(x, M, preferred_element_type=jnp.float32)
    xc = x - mean
    var = jnp.dot(xc * xc, M, preferred_element_type=jnp.float32) + 1e-9
    y = xc * sc_ref[...] * lax.rsqrt(var) + of_ref[...]
    out_ref[...] = _matT(y, wa_ref[...]) + ba_ref[...] + fdst_ref[...]


def _final(agg, fdst, W_agg, b_agg, scale2, offset2):
    R = 1000
    GN = N // R
    return pl.pallas_call(
        _final_body,
        grid=(GN,),
        in_specs=[
            pl.BlockSpec((R, HHALF), lambda i: (i, 0)),
            pl.BlockSpec((R, HHALF), lambda i: (GN + i, 0)),
            pl.BlockSpec((R, HF), lambda i: (i, 0)),
            pl.BlockSpec((HF, HF), lambda i: (0, 0)),
            pl.BlockSpec((1, HF), lambda i: (0, 0)),
            pl.BlockSpec((1, HF), lambda i: (0, 0)),
            pl.BlockSpec((1, HF), lambda i: (0, 0)),
        ],
        out_specs=pl.BlockSpec((R, HF), lambda i: (i, 0)),
        out_shape=jax.ShapeDtypeStruct((N, HF), jnp.float32),
    )(agg, agg, fdst, W_agg, b_agg, scale2, offset2)


def kernel(feat_src, edge_index, feat_edge, W_src, W_attn_src, W_attn_dst,
           W_attn_edge, scale, offset, W_agg, b_agg, W_dst, b_dst):
    W_cat = jnp.concatenate([W_attn_src, W_attn_dst], axis=0)
    fcS, attn_tbl, fdst, aeT = _node_proj(
        feat_src, W_src, W_cat, W_dst, b_dst.reshape(1, HF),
        feat_edge, W_attn_edge)
    src = edge_index[0]
    dst = edge_index[1]
    e_dbg = attn_tbl[:, :H][src] + attn_tbl[:, H:][dst] + aeT.T
    a_edge = jnp.where(e_dbg > 0, e_dbg, e_dbg * 0.2).T.reshape(H * E)
    zeros = jnp.zeros((NPT, HHALF), jnp.float32)
    dst4d = dst.reshape(NSUB, NSK, NC2, K)
    agg = _sc_edge_kernel(fcS, a_edge, src, dst4d, zeros)
    rst = _final(agg, fdst, W_agg, b_agg.reshape(1, HF),
                 scale.reshape(1, HF), offset.reshape(1, HF))
    return rst.reshape(N, H, F)


# speedup vs baseline: 3.6710x; 3.6710x over previous
import functools

import jax
import jax.numpy as jnp
from jax import lax
from jax.experimental import pallas as pl
from jax.experimental.pallas import tpu as pltpu
from jax.experimental.pallas import tpu_sc as plsc

N = 10000
E = 320000
D = 128
DE = 16
H = 4
F = 64
HF = H * F
HHALF = HF // 2

NSUB = 16
EPER = E // NSUB
K = 80
SK = 2000
NSK = EPER // SK
NC2 = SK // K
NQ = NC2 // 4
NPT = 624
NTAIL = N - NSUB * NPT

NW = 32
EW = E // NW


def _matT(x, w):
    return lax.dot_general(x, w, (((1,), (1,)), ((), ())),
                           preferred_element_type=jnp.float32)


def _proj_body(x_ref, ws_ref, wc_ref, wd_ref, bd_ref, fe_ref, we_ref,
               fcs_ref, attn_ref, fdst_ref, aeT_ref):
    x = x_ref[...]
    fcs_ref[...] = _matT(x, ws_ref[...])
    attn_ref[...] = _matT(x, wc_ref[...])
    fdst_ref[...] = _matT(x, wd_ref[...]) + bd_ref[...]
    aeT_ref[...] = lax.dot_general(
        we_ref[...], fe_ref[...], (((1,), (1,)), ((), ())),
        preferred_element_type=jnp.float32)


def _node_proj(feat_src, W_src, W_cat, W_dst, b_dst, feat_edge, W_attn_edge):
    R = 1000
    GN = N // R
    EB = E // (2 * GN)
    return pl.pallas_call(
        _proj_body,
        grid=(2, GN),
        in_specs=[
            pl.BlockSpec((R, D), lambda h, i: (i, 0)),
            pl.BlockSpec((HHALF, D), lambda h, i: (h, 0)),
            pl.BlockSpec((2 * H, D), lambda h, i: (0, 0)),
            pl.BlockSpec((HF, D), lambda h, i: (0, 0)),
            pl.BlockSpec((1, HF), lambda h, i: (0, 0)),
            pl.BlockSpec((EB, DE), lambda h, i: (h * GN + i, 0)),
            pl.BlockSpec((H, DE), lambda h, i: (0, 0)),
        ],
        out_specs=[
            pl.BlockSpec((R, HHALF), lambda h, i: (h * GN + i, 0)),
            pl.BlockSpec((R, 2 * H), lambda h, i: (i, 0)),
            pl.BlockSpec((R, HF), lambda h, i: (i, 0)),
            pl.BlockSpec((H, EB), lambda h, i: (0, h * GN + i)),
        ],
        out_shape=[
            jax.ShapeDtypeStruct((2 * N, HHALF), jnp.float32),
            jax.ShapeDtypeStruct((N, 2 * H), jnp.float32),
            jax.ShapeDtypeStruct((N, HF), jnp.float32),
            jax.ShapeDtypeStruct((H, E), jnp.float32),
        ],
    )(feat_src, W_src, W_cat, W_dst, b_dst, feat_edge, W_attn_edge)


def _sc_attn_kernel(attn_flat, aeT, src, dst):
    mesh = plsc.VectorSubcoreMesh(core_axis_name="c", subcore_axis_name="s")

    @functools.partial(
        pl.kernel,
        out_type=jax.ShapeDtypeStruct((H * E,), jnp.float32),
        mesh=mesh,
        scratch_types=[
            pltpu.VMEM((N * 2 * H,), jnp.float32),
            pltpu.VMEM((EW,), jnp.int32),
            pltpu.VMEM((EW,), jnp.int32),
            [pltpu.VMEM((EW,), jnp.float32) for _ in range(2)],
        ],
        compiler_params=pltpu.CompilerParams(needs_layout_passes=False),
    )
    def k(attn_hbm, aeT_hbm, src_hbm, dst_hbm, a_hbm,
          tbl, srcv, dstv, aebs):
        c = lax.axis_index("c")
        s = lax.axis_index("s")
        w = s * 2 + c
        base = w * EW
        pltpu.sync_copy(attn_hbm, tbl)
        pltpu.sync_copy(src_hbm.at[pl.ds(base, EW)], srcv)
        pltpu.sync_copy(dst_hbm.at[pl.ds(base, EW)], dstv)

        for hp in range(2):
            for hl in range(2):
                h = 2 * hp + hl
                pltpu.sync_copy(aeT_hbm.at[pl.ds(h * E + base, EW)],
                                aebs[hl])

            def attn16(off):
                sidx = srcv[pl.ds(off, 16)] * (2 * H)
                didx = dstv[pl.ds(off, 16)] * (2 * H)
                for hl in range(2):
                    h = 2 * hp + hl
                    asrc = plsc.load_gather(tbl, [sidx + h])
                    adst = plsc.load_gather(tbl, [didx + (H + h)])
                    e = asrc + adst + aebs[hl][pl.ds(off, 16)]
                    aebs[hl][pl.ds(off, 16)] = jnp.where(e > 0, e, e * 0.2)

            def group(g, carry2):
                attn16(g * 32)
                attn16(g * 32 + 16)
                return carry2

            lax.fori_loop(0, EW // 32, group, 0)
            attn16(EW - 16)
            for hl in range(2):
                h = 2 * hp + hl
                pltpu.sync_copy(aebs[hl], a_hbm.at[pl.ds(h * E + base, EW)])

    return k(attn_flat, aeT, src, dst)


def _sc_edge_kernel(fcS, a_hbm_in, src, dst4d, zeros_hbm):
    mesh = plsc.VectorSubcoreMesh(core_axis_name="c", subcore_axis_name="s")

    @functools.partial(
        pl.kernel,
        out_type=jax.ShapeDtypeStruct((2 * N, HHALF), jnp.float32),
        mesh=mesh,
        scratch_types=[
            [pltpu.VMEM((K, HHALF), jnp.float32) for _ in range(4)],
            pltpu.VMEM((SK,), jnp.int32),
            pltpu.VMEM((NC2, K), jnp.int32),
            [pltpu.VMEM((K,), jnp.float32) for _ in range(4)],
            [pltpu.VMEM((K,), jnp.float32) for _ in range(4)],
            pltpu.VMEM_SHARED((N, HHALF), jnp.float32),
            [pltpu.SemaphoreType.DMA for _ in range(4)],
            [pltpu.SemaphoreType.DMA for _ in range(4)],
            pltpu.SemaphoreType.DMA,
        ],
        compiler_params=pltpu.CompilerParams(needs_layout_passes=False),
    )
    def k(fcs_hbm, a_hbm, src_hbm, dst4d_hbm, z_hbm, agg_hbm,
          rows, srcv, dstv2d, ablo, abhi, accum, gsems, ssems, isem):
        c = lax.axis_index("c")
        s = lax.axis_index("s")
        h0 = 2 * c
        cN = c * N
        rbase = s * NPT

        pltpu.sync_copy(z_hbm, accum.at[pl.ds(rbase, NPT)])

        @pl.when(s == 0)
        def _():
            pltpu.sync_copy(z_hbm.at[pl.ds(0, NTAIL)],
                            accum.at[pl.ds(NSUB * NPT, NTAIL)])

        plsc.subcore_barrier()

        def compute(b, cd):
            def mul16(off):
                a0 = ablo[b][pl.ds(off, 16)]
                a1 = abhi[b][pl.ds(off, 16)]
                for j in range(16):
                    bidx = jnp.full((16,), j, jnp.int32)
                    erow = off + j
                    for hl, av in ((0, a0), (1, a1)):
                        bc = av.at[bidx].get(mode="promise_in_bounds")
                        for q in range(F // 16):
                            cs = hl * F + q * 16
                            rows[b][erow, pl.ds(cs, 16)] = (
                                rows[b][erow, pl.ds(cs, 16)] * bc)

            def group(g, carry):
                mul16(g * 32)
                mul16(g * 32 + 16)
                return carry

            lax.fori_loop(0, K // 32, group, 0)
            mul16(K - 16)

        def g_start(b, cd, base):
            pltpu.async_copy(fcs_hbm.at[srcv.at[pl.ds(cd * K, K)]],
                             rows[b], gsems[b])
            ebase = base + cd * K
            pltpu.async_copy(a_hbm.at[pl.ds(h0 * E + ebase, K)], ablo[b],
                             gsems[b])
            pltpu.async_copy(a_hbm.at[pl.ds((h0 + 1) * E + ebase, K)],
                             abhi[b], gsems[b])

        def g_wait(b, cd, base):
            pltpu.make_async_copy(fcs_hbm.at[srcv.at[pl.ds(cd * K, K)]],
                                  rows[b], gsems[b]).wait()
            ebase = base + cd * K
            pltpu.make_async_copy(a_hbm.at[pl.ds(h0 * E + ebase, K)],
                                  ablo[b], gsems[b]).wait()
            pltpu.make_async_copy(a_hbm.at[pl.ds((h0 + 1) * E + ebase, K)],
                                  abhi[b], gsems[b]).wait()

        def sc_start(b, cd):
            pltpu.async_copy(rows[b], accum.at[dstv2d.at[cd]], ssems[b],
                             add=True)

        def sc_wait(b, cd):
            pltpu.make_async_copy(rows[b], accum.at[dstv2d.at[cd]],
                                  ssems[b]).wait()

        def sk_body(j, carry):
            base = s * EPER + j * SK
            pltpu.async_copy(src_hbm.at[pl.ds(base, SK)], srcv, isem)
            pltpu.async_copy(dst4d_hbm.at[s, j], dstv2d, isem)
            pltpu.make_async_copy(src_hbm.at[pl.ds(base, SK)], srcv,
                                  isem).wait()
            pltpu.make_async_copy(dst4d_hbm.at[s, j], dstv2d, isem).wait()

            def adj(g, carry2):
                srcv[pl.ds(g * 16, 16)] = srcv[pl.ds(g * 16, 16)] + cN
                return carry2

            lax.fori_loop(0, SK // 16, adj, 0)
            g_start(0, 0, base)
            g_start(1, 1, base)

            def quad(p, carry2):
                c0 = 4 * p
                for t in range(4):
                    cd = c0 + t
                    b = t
                    b2 = (t + 2) % 4
                    if t >= 2:
                        sc_wait(b2, cd - 2)
                    else:
                        @pl.when(p > 0)
                        def _():
                            sc_wait(b2, cd - 2)
                    if t < 3:
                        g_start(b2, cd + 2, base)
                    else:
                        @pl.when(p < NQ - 1)
                        def _():
                            g_start(b2, cd + 2, base)
                    g_wait(b, cd, base)
                    compute(b, cd)
                    sc_start(b, cd)
                return carry2

            lax.fori_loop(0, NQ, quad, 0)
            sc_wait(2, NC2 - 3)
            g_wait(0, NC2 - 1, base)
            compute(0, NC2 - 1)
            sc_start(0, NC2 - 1)
            sc_wait(3, NC2 - 2)
            sc_wait(0, NC2 - 1)
            return carry

        lax.fori_loop(0, NSK, sk_body, 0)
        plsc.subcore_barrier()
        pltpu.sync_copy(accum.at[pl.ds(rbase, NPT)],
                        agg_hbm.at[pl.ds(cN + rbase, NPT)])

        @pl.when(s == 0)
        def _():
            pltpu.sync_copy(accum.at[pl.ds(NSUB * NPT, NTAIL)],
                            agg_hbm.at[pl.ds(cN + NSUB * NPT, NTAIL)])

    return k(fcS, a_hbm_in, src, dst4d, zeros_hbm)


def _final_body(a0_ref, a1_ref, fdst_ref, wa_ref, ba_ref, sc_ref, of_ref,
                out_ref):
    x = jnp.concatenate([a0_ref[...], a1_ref[...]], axis=1)
    r = lax.broadcasted_iota(jnp.int32, (HF, HF), 0) // F
    cc = lax.broadcasted_iota(jnp.int32, (HF, HF), 1) // F
    M = jnp.where(r == cc, jnp.float32(1.0 / F), jnp.float32(0.0))
    mean = jnp.dot(x, M, preferred_element_type=jnp.float32)
    xc = x - mean
    var = jnp.dot(xc * xc, M, preferred_element_type=jnp.float32) + 1e-9
    y = xc * sc_ref[...] * lax.rsqrt(var) + of_ref[...]
    out_ref[...] = _matT(y, wa_ref[...]) + ba_ref[...] + fdst_ref[...]


def _final(agg, fdst, W_agg, b_agg, scale2, offset2):
    R = 1000
    GN = N // R
    return pl.pallas_call(
        _final_body,
        grid=(GN,),
        in_specs=[
            pl.BlockSpec((R, HHALF), lambda i: (i, 0)),
            pl.BlockSpec((R, HHALF), lambda i: (GN + i, 0)),
            pl.BlockSpec((R, HF), lambda i: (i, 0)),
            pl.BlockSpec((HF, HF), lambda i: (0, 0)),
            pl.BlockSpec((1, HF), lambda i: (0, 0)),
            pl.BlockSpec((1, HF), lambda i: (0, 0)),
            pl.BlockSpec((1, HF), lambda i: (0, 0)),
        ],
        out_specs=pl.BlockSpec((R, HF), lambda i: (i, 0)),
        out_shape=jax.ShapeDtypeStruct((N, HF), jnp.float32),
    )(agg, agg, fdst, W_agg, b_agg, scale2, offset2)


def kernel(feat_src, edge_index, feat_edge, W_src, W_attn_src, W_attn_dst,
           W_attn_edge, scale, offset, W_agg, b_agg, W_dst, b_dst):
    W_cat = jnp.concatenate([W_attn_src, W_attn_dst], axis=0)
    fcS, attn_tbl, fdst, aeT = _node_proj(
        feat_src, W_src, W_cat, W_dst, b_dst.reshape(1, HF),
        feat_edge, W_attn_edge)
    src = edge_index[0]
    dst = edge_index[1]
    a_edge = _sc_attn_kernel(attn_tbl.reshape(N * 2 * H),
                             aeT.reshape(H * E), src, dst)
    zeros = jnp.zeros((NPT, HHALF), jnp.float32)
    dst4d = dst.reshape(NSUB, NSK, NC2, K)
    agg = _sc_edge_kernel(fcS, a_edge, src, dst4d, zeros)
    rst = _final(agg, fdst, W_agg, b_agg.reshape(1, HF),
                 scale.reshape(1, HF), offset.reshape(1, HF))
    return rst.reshape(N, H, F)


# speedup vs baseline: 3.7887x; 1.0321x over previous
import functools

import jax
import jax.numpy as jnp
from jax import lax
from jax.experimental import pallas as pl
from jax.experimental.pallas import tpu as pltpu
from jax.experimental.pallas import tpu_sc as plsc

N = 10000
E = 320000
D = 128
DE = 16
H = 4
F = 64
HF = H * F
HHALF = HF // 2

NSUB = 16
EPER = E // NSUB
K = 80
SK = 2000
NSK = EPER // SK
NC2 = SK // K
NQ = NC2 // 4
NPT = 624
NTAIL = N - NSUB * NPT

NW = 32
EW = E // NW


def _matT(x, w):
    return lax.dot_general(x, w, (((1,), (1,)), ((), ())),
                           preferred_element_type=jnp.float32)


def _proj_body(x_ref, ws_ref, wc_ref, wd_ref, bd_ref, fe_ref, we_ref,
               fcs_ref, attn_ref, fdst_ref, aeT_ref):
    x = x_ref[...]
    fcs_ref[...] = _matT(x, ws_ref[...])
    attn_ref[...] = _matT(x, wc_ref[...])
    fdst_ref[...] = _matT(x, wd_ref[...]) + bd_ref[...]
    aeT_ref[...] = lax.dot_general(
        we_ref[...], fe_ref[...], (((1,), (1,)), ((), ())),
        preferred_element_type=jnp.float32)


def _node_proj(feat_src, W_src, W_cat, W_dst, b_dst, feat_edge, W_attn_edge):
    R = 1000
    GN = N // R
    EB = E // (2 * GN)
    return pl.pallas_call(
        _proj_body,
        grid=(2, GN),
        in_specs=[
            pl.BlockSpec((R, D), lambda h, i: (i, 0)),
            pl.BlockSpec((HHALF, D), lambda h, i: (h, 0)),
            pl.BlockSpec((2 * H, D), lambda h, i: (0, 0)),
            pl.BlockSpec((HF, D), lambda h, i: (0, 0)),
            pl.BlockSpec((1, HF), lambda h, i: (0, 0)),
            pl.BlockSpec((EB, DE), lambda h, i: (h * GN + i, 0)),
            pl.BlockSpec((H, DE), lambda h, i: (0, 0)),
        ],
        out_specs=[
            pl.BlockSpec((R, HHALF), lambda h, i: (h * GN + i, 0)),
            pl.BlockSpec((R, 2 * H), lambda h, i: (i, 0)),
            pl.BlockSpec((R, HF), lambda h, i: (i, 0)),
            pl.BlockSpec((H, EB), lambda h, i: (0, h * GN + i)),
        ],
        out_shape=[
            jax.ShapeDtypeStruct((2 * N, HHALF), jnp.float32),
            jax.ShapeDtypeStruct((N, 2 * H), jnp.float32),
            jax.ShapeDtypeStruct((N, HF), jnp.float32),
            jax.ShapeDtypeStruct((H, E), jnp.float32),
        ],
    )(feat_src, W_src, W_cat, W_dst, b_dst, feat_edge, W_attn_edge)


def _sc_attn_kernel(attn_flat, aeT, src, dst):
    mesh = plsc.VectorSubcoreMesh(core_axis_name="c", subcore_axis_name="s")

    @functools.partial(
        pl.kernel,
        out_type=jax.ShapeDtypeStruct((H * E,), jnp.float32),
        mesh=mesh,
        scratch_types=[
            pltpu.VMEM((N * 2 * H,), jnp.float32),
            pltpu.VMEM((EW,), jnp.int32),
            pltpu.VMEM((EW,), jnp.int32),
            [pltpu.VMEM((EW,), jnp.float32) for _ in range(2)],
        ],
        compiler_params=pltpu.CompilerParams(needs_layout_passes=False),
    )
    def k(attn_hbm, aeT_hbm, src_hbm, dst_hbm, a_hbm,
          tbl, srcv, dstv, aebs):
        c = lax.axis_index("c")
        s = lax.axis_index("s")
        w = s * 2 + c
        base = w * EW
        pltpu.sync_copy(attn_hbm, tbl)
        pltpu.sync_copy(src_hbm.at[pl.ds(base, EW)], srcv)
        pltpu.sync_copy(dst_hbm.at[pl.ds(base, EW)], dstv)

        for hp in range(2):
            for hl in range(2):
                h = 2 * hp + hl
                pltpu.sync_copy(aeT_hbm.at[pl.ds(h * E + base, EW)],
                                aebs[hl])

            def group(g, carry2):
                off = g * 16
                sidx = srcv[pl.ds(off, 16)] * (2 * H)
                didx = dstv[pl.ds(off, 16)] * (2 * H)
                for hl in range(2):
                    h = 2 * hp + hl
                    asrc = plsc.load_gather(tbl, [sidx + h])
                    adst = plsc.load_gather(tbl, [didx + (H + h)])
                    e = asrc + adst + aebs[hl][pl.ds(off, 16)]
                    aebs[hl][pl.ds(off, 16)] = jnp.where(e > 0, e, e * 0.2)
                return carry2

            lax.fori_loop(0, EW // 16, group, 0)
            for hl in range(2):
                h = 2 * hp + hl
                pltpu.sync_copy(aebs[hl], a_hbm.at[pl.ds(h * E + base, EW)])

    return k(attn_flat, aeT, src, dst)


def _sc_edge_kernel(fcS, a_hbm_in, src, dst4d, zeros_hbm):
    mesh = plsc.VectorSubcoreMesh(core_axis_name="c", subcore_axis_name="s")

    @functools.partial(
        pl.kernel,
        out_type=jax.ShapeDtypeStruct((2 * N, HHALF), jnp.float32),
        mesh=mesh,
        scratch_types=[
            [pltpu.VMEM((K, HHALF), jnp.float32) for _ in range(4)],
            pltpu.VMEM((SK,), jnp.int32),
            pltpu.VMEM((NC2, K), jnp.int32),
            [pltpu.VMEM((K,), jnp.float32) for _ in range(4)],
            [pltpu.VMEM((K,), jnp.float32) for _ in range(4)],
            pltpu.VMEM_SHARED((N, HHALF), jnp.float32),
            [pltpu.SemaphoreType.DMA for _ in range(4)],
            [pltpu.SemaphoreType.DMA for _ in range(4)],
            pltpu.SemaphoreType.DMA,
        ],
        compiler_params=pltpu.CompilerParams(needs_layout_passes=False),
    )
    def k(fcs_hbm, a_hbm, src_hbm, dst4d_hbm, z_hbm, agg_hbm,
          rows, srcv, dstv2d, ablo, abhi, accum, gsems, ssems, isem):
        c = lax.axis_index("c")
        s = lax.axis_index("s")
        h0 = 2 * c
        cN = c * N
        rbase = s * NPT

        pltpu.sync_copy(z_hbm, accum.at[pl.ds(rbase, NPT)])

        @pl.when(s == 0)
        def _():
            pltpu.sync_copy(z_hbm.at[pl.ds(0, NTAIL)],
                            accum.at[pl.ds(NSUB * NPT, NTAIL)])

        plsc.subcore_barrier()

        def compute(b, cd):
            def group(g, carry):
                off = g * 16
                a0 = ablo[b][pl.ds(off, 16)]
                a1 = abhi[b][pl.ds(off, 16)]
                for j in range(16):
                    bidx = jnp.full((16,), j, jnp.int32)
                    erow = off + j
                    for hl, av in ((0, a0), (1, a1)):
                        bc = av.at[bidx].get(mode="promise_in_bounds")
                        for q in range(F // 16):
                            cs = hl * F + q * 16
                            rows[b][erow, pl.ds(cs, 16)] = (
                                rows[b][erow, pl.ds(cs, 16)] * bc)
                return carry

            lax.fori_loop(0, K // 16, group, 0)

        def g_start(b, cd, base):
            pltpu.async_copy(fcs_hbm.at[srcv.at[pl.ds(cd * K, K)]],
                             rows[b], gsems[b])
            ebase = base + cd * K
            pltpu.async_copy(a_hbm.at[pl.ds(h0 * E + ebase, K)], ablo[b],
                             gsems[b])
            pltpu.async_copy(a_hbm.at[pl.ds((h0 + 1) * E + ebase, K)],
                             abhi[b], gsems[b])

        def g_wait(b, cd, base):
            pltpu.make_async_copy(fcs_hbm.at[srcv.at[pl.ds(cd * K, K)]],
                                  rows[b], gsems[b]).wait()
            ebase = base + cd * K
            pltpu.make_async_copy(a_hbm.at[pl.ds(h0 * E + ebase, K)],
                                  ablo[b], gsems[b]).wait()
            pltpu.make_async_copy(a_hbm.at[pl.ds((h0 + 1) * E + ebase, K)],
                                  abhi[b], gsems[b]).wait()

        def sc_start(b, cd):
            pltpu.async_copy(rows[b], accum.at[dstv2d.at[cd]], ssems[b],
                             add=True)

        def sc_wait(b, cd):
            pltpu.make_async_copy(rows[b], accum.at[dstv2d.at[cd]],
                                  ssems[b]).wait()

        def sk_body(j, carry):
            base = s * EPER + j * SK
            pltpu.async_copy(src_hbm.at[pl.ds(base, SK)], srcv, isem)
            pltpu.async_copy(dst4d_hbm.at[s, j], dstv2d, isem)
            pltpu.make_async_copy(src_hbm.at[pl.ds(base, SK)], srcv,
                                  isem).wait()
            pltpu.make_async_copy(dst4d_hbm.at[s, j], dstv2d, isem).wait()

            def adj(g, carry2):
                srcv[pl.ds(g * 16, 16)] = srcv[pl.ds(g * 16, 16)] + cN
                return carry2

            lax.fori_loop(0, SK // 16, adj, 0)
            g_start(0, 0, base)
            g_start(1, 1, base)

            def quad(p, carry2):
                c0 = 4 * p
                for t in range(4):
                    cd = c0 + t
                    b = t
                    b2 = (t + 2) % 4
                    if t >= 2:
                        sc_wait(b2, cd - 2)
                    else:
                        @pl.when(p > 0)
                        def _():
                            sc_wait(b2, cd - 2)
                    if t < 3:
                        g_start(b2, cd + 2, base)
                    else:
                        @pl.when(p < NQ - 1)
                        def _():
                            g_start(b2, cd + 2, base)
                    g_wait(b, cd, base)
                    compute(b, cd)
                    sc_start(b, cd)
                return carry2

            lax.fori_loop(0, NQ, quad, 0)
            sc_wait(2, NC2 - 3)
            g_wait(0, NC2 - 1, base)
            compute(0, NC2 - 1)
            sc_start(0, NC2 - 1)
            sc_wait(3, NC2 - 2)
            sc_wait(0, NC2 - 1)
            return carry

        lax.fori_loop(0, NSK, sk_body, 0)
        plsc.subcore_barrier()
        pltpu.sync_copy(accum.at[pl.ds(rbase, NPT)],
                        agg_hbm.at[pl.ds(cN + rbase, NPT)])

        @pl.when(s == 0)
        def _():
            pltpu.sync_copy(accum.at[pl.ds(NSUB * NPT, NTAIL)],
                            agg_hbm.at[pl.ds(cN + NSUB * NPT, NTAIL)])

    return k(fcS, a_hbm_in, src, dst4d, zeros_hbm)


def _final_body(a0_ref, a1_ref, fdst_ref, wa_ref, ba_ref, sc_ref, of_ref,
                out_ref):
    x = jnp.concatenate([a0_ref[...], a1_ref[...]], axis=1)
    r = lax.broadcasted_iota(jnp.int32, (HF, HF), 0) // F
    cc = lax.broadcasted_iota(jnp.int32, (HF, HF), 1) // F
    M = jnp.where(r == cc, jnp.float32(1.0 / F), jnp.float32(0.0))
    mean = jnp.dot(x, M, preferred_element_type=jnp.float32)
    xc = x - mean
    var = jnp.dot(xc * xc, M, preferred_element_type=jnp.float32) + 1e-9
    y = xc * sc_ref[...] * lax.rsqrt(var) + of_ref[...]
    out = _matT(y, wa_ref[...]) + ba_ref[...] + fdst_ref[...]
    out_ref[...] = out.reshape(out.shape[0], H, F)


def _final(agg, fdst, W_agg, b_agg, scale2, offset2):
    R = 1000
    GN = N // R
    return pl.pallas_call(
        _final_body,
        grid=(GN,),
        in_specs=[
            pl.BlockSpec((R, HHALF), lambda i: (i, 0)),
            pl.BlockSpec((R, HHALF), lambda i: (GN + i, 0)),
            pl.BlockSpec((R, HF), lambda i: (i, 0)),
            pl.BlockSpec((HF, HF), lambda i: (0, 0)),
            pl.BlockSpec((1, HF), lambda i: (0, 0)),
            pl.BlockSpec((1, HF), lambda i: (0, 0)),
            pl.BlockSpec((1, HF), lambda i: (0, 0)),
        ],
        out_specs=pl.BlockSpec((R, H, F), lambda i: (i, 0, 0)),
        out_shape=jax.ShapeDtypeStruct((N, H, F), jnp.float32),
    )(agg, agg, fdst, W_agg, b_agg, scale2, offset2)


def kernel(feat_src, edge_index, feat_edge, W_src, W_attn_src, W_attn_dst,
           W_attn_edge, scale, offset, W_agg, b_agg, W_dst, b_dst):
    W_cat = jnp.concatenate([W_attn_src, W_attn_dst], axis=0)
    fcS, attn_tbl, fdst, aeT = _node_proj(
        feat_src, W_src, W_cat, W_dst, b_dst.reshape(1, HF),
        feat_edge, W_attn_edge)
    src = edge_index[0]
    dst = edge_index[1]
    a_edge = _sc_attn_kernel(attn_tbl.reshape(N * 2 * H),
                             aeT.reshape(H * E), src, dst)
    zeros = jnp.zeros((NPT, HHALF), jnp.float32)
    dst4d = dst.reshape(NSUB, NSK, NC2, K)
    agg = _sc_edge_kernel(fcS, a_edge, src, dst4d, zeros)
    return _final(agg, fdst, W_agg, b_agg.reshape(1, HF),
                  scale.reshape(1, HF), offset.reshape(1, HF))


# speedup vs baseline: 3.8896x; 1.0266x over previous
import functools

import jax
import jax.numpy as jnp
from jax import lax
from jax.experimental import pallas as pl
from jax.experimental.pallas import tpu as pltpu
from jax.experimental.pallas import tpu_sc as plsc

N = 10000
E = 320000
D = 128
DE = 16
H = 4
F = 64
HF = H * F
HHALF = HF // 2

NSUB = 16
EPER = E // NSUB
K = 80
SK = 2000
NSK = EPER // SK
NC2 = SK // K
NQ = NC2 // 4
NPT = 624
NTAIL = N - NSUB * NPT

NW = 32
EW = E // NW


def _matT(x, w):
    return lax.dot_general(x, w, (((1,), (1,)), ((), ())),
                           preferred_element_type=jnp.float32)


def _proj_body(x_ref, ws_ref, wc_ref, wd_ref, bd_ref, fe_ref, we_ref,
               fcs_ref, attn_ref, fdst_ref, aeT_ref):
    x = x_ref[...]
    fcs_ref[...] = _matT(x, ws_ref[...])
    attn_ref[...] = _matT(x, wc_ref[...])
    fdst_ref[...] = _matT(x, wd_ref[...]) + bd_ref[...]
    aeT_ref[...] = lax.dot_general(
        we_ref[...], fe_ref[...], (((1,), (1,)), ((), ())),
        preferred_element_type=jnp.float32)


def _node_proj(feat_src, W_src, W_cat, W_dst, b_dst, feat_edge, W_attn_edge):
    R = 1000
    GN = N // R
    EB = E // (2 * GN)
    return pl.pallas_call(
        _proj_body,
        grid=(2, GN),
        in_specs=[
            pl.BlockSpec((R, D), lambda h, i: (i, 0)),
            pl.BlockSpec((HHALF, D), lambda h, i: (h, 0)),
            pl.BlockSpec((2 * H, D), lambda h, i: (0, 0)),
            pl.BlockSpec((HF, D), lambda h, i: (0, 0)),
            pl.BlockSpec((1, HF), lambda h, i: (0, 0)),
            pl.BlockSpec((EB, DE), lambda h, i: (h * GN + i, 0)),
            pl.BlockSpec((H, DE), lambda h, i: (0, 0)),
        ],
        out_specs=[
            pl.BlockSpec((R, HHALF), lambda h, i: (h * GN + i, 0)),
            pl.BlockSpec((R, 2 * H), lambda h, i: (i, 0)),
            pl.BlockSpec((R, HF), lambda h, i: (i, 0)),
            pl.BlockSpec((H, EB), lambda h, i: (0, h * GN + i)),
        ],
        out_shape=[
            jax.ShapeDtypeStruct((2 * N, HHALF), jnp.float32),
            jax.ShapeDtypeStruct((N, 2 * H), jnp.float32),
            jax.ShapeDtypeStruct((N, HF), jnp.float32),
            jax.ShapeDtypeStruct((H, E), jnp.float32),
        ],
    )(feat_src, W_src, W_cat, W_dst, b_dst, feat_edge, W_attn_edge)


def _sc_attn_kernel(attn_flat, aeT, src, dst):
    mesh = plsc.VectorSubcoreMesh(core_axis_name="c", subcore_axis_name="s")

    @functools.partial(
        pl.kernel,
        out_type=jax.ShapeDtypeStruct((H * E,), jnp.float32),
        mesh=mesh,
        scratch_types=[
            pltpu.VMEM((N * 2 * H,), jnp.float32),
            pltpu.VMEM((EW,), jnp.int32),
            pltpu.VMEM((EW,), jnp.int32),
            [pltpu.VMEM((EW,), jnp.float32) for _ in range(2)],
        ],
        compiler_params=pltpu.CompilerParams(needs_layout_passes=False),
    )
    def k(attn_hbm, aeT_hbm, src_hbm, dst_hbm, a_hbm,
          tbl, srcv, dstv, aebs):
        c = lax.axis_index("c")
        s = lax.axis_index("s")
        w = s * 2 + c
        base = w * EW
        pltpu.sync_copy(attn_hbm, tbl)
        pltpu.sync_copy(src_hbm.at[pl.ds(base, EW)], srcv)
        pltpu.sync_copy(dst_hbm.at[pl.ds(base, EW)], dstv)

        for hp in range(2):
            for hl in range(2):
                h = 2 * hp + hl
                pltpu.sync_copy(aeT_hbm.at[pl.ds(h * E + base, EW)],
                                aebs[hl])

            def group(g, carry2):
                off = g * 16
                sidx = srcv[pl.ds(off, 16)] * (2 * H)
                didx = dstv[pl.ds(off, 16)] * (2 * H)
                for hl in range(2):
                    h = 2 * hp + hl
                    asrc = plsc.load_gather(tbl, [sidx + h])
                    adst = plsc.load_gather(tbl, [didx + (H + h)])
                    e = asrc + adst + aebs[hl][pl.ds(off, 16)]
                    aebs[hl][pl.ds(off, 16)] = jnp.where(e > 0, e, e * 0.2)
                return carry2

            lax.fori_loop(0, EW // 16, group, 0)
            for hl in range(2):
                h = 2 * hp + hl
                pltpu.sync_copy(aebs[hl], a_hbm.at[pl.ds(h * E + base, EW)])

    return k(attn_flat, aeT, src, dst)


def _sc_edge_kernel(fcS, a_hbm_in, src, dst4d, zeros_hbm):
    mesh = plsc.VectorSubcoreMesh(core_axis_name="c", subcore_axis_name="s")

    @functools.partial(
        pl.kernel,
        out_type=jax.ShapeDtypeStruct((2 * N, HHALF), jnp.float32),
        mesh=mesh,
        scratch_types=[
            [pltpu.VMEM((K, HHALF), jnp.float32) for _ in range(4)],
            pltpu.VMEM((SK,), jnp.int32),
            pltpu.VMEM((NC2, K), jnp.int32),
            [pltpu.VMEM((K,), jnp.float32) for _ in range(4)],
            [pltpu.VMEM((K,), jnp.float32) for _ in range(4)],
            pltpu.VMEM_SHARED((N, HHALF), jnp.float32),
            [pltpu.SemaphoreType.DMA for _ in range(4)],
            [pltpu.SemaphoreType.DMA for _ in range(4)],
            pltpu.SemaphoreType.DMA,
        ],
        compiler_params=pltpu.CompilerParams(needs_layout_passes=False),
    )
    def k(fcs_hbm, a_hbm, src_hbm, dst4d_hbm, z_hbm, agg_hbm,
          rows, srcv, dstv2d, ablo, abhi, accum, gsems, ssems, isem):
        c = lax.axis_index("c")
        s = lax.axis_index("s")
        h0 = 2 * c
        cN = c * N
        rbase = s * NPT

        pltpu.sync_copy(z_hbm, accum.at[pl.ds(rbase, NPT)])

        @pl.when(s == 0)
        def _():
            pltpu.sync_copy(z_hbm.at[pl.ds(0, NTAIL)],
                            accum.at[pl.ds(NSUB * NPT, NTAIL)])

        plsc.subcore_barrier()

        def compute(b, cd):
            def group(g, carry):
                off = g * 16
                a0 = ablo[b][pl.ds(off, 16)]
                a1 = abhi[b][pl.ds(off, 16)]
                for j in range(16):
                    bidx = jnp.full((16,), j, jnp.int32)
                    erow = off + j
                    for hl, av in ((0, a0), (1, a1)):
                        bc = av.at[bidx].get(mode="promise_in_bounds")
                        for q in range(F // 16):
                            cs = hl * F + q * 16
                            rows[b][erow, pl.ds(cs, 16)] = (
                                rows[b][erow, pl.ds(cs, 16)] * bc)
                return carry

            lax.fori_loop(0, K // 16, group, 0)

        def g_start(b, cd, base):
            pltpu.async_copy(fcs_hbm.at[srcv.at[pl.ds(cd * K, K)]],
                             rows[b], gsems[b])
            ebase = base + cd * K
            pltpu.async_copy(a_hbm.at[pl.ds(h0 * E + ebase, K)], ablo[b],
                             gsems[b])
            pltpu.async_copy(a_hbm.at[pl.ds((h0 + 1) * E + ebase, K)],
                             abhi[b], gsems[b])

        def g_wait(b, cd, base):
            pltpu.make_async_copy(fcs_hbm.at[srcv.at[pl.ds(cd * K, K)]],
                                  rows[b], gsems[b]).wait()
            ebase = base + cd * K
            pltpu.make_async_copy(a_hbm.at[pl.ds(h0 * E + ebase, K)],
                                  ablo[b], gsems[b]).wait()
            pltpu.make_async_copy(a_hbm.at[pl.ds((h0 + 1) * E + ebase, K)],
                                  abhi[b], gsems[b]).wait()

        def sc_start(b, cd):
            pltpu.async_copy(rows[b], accum.at[dstv2d.at[cd]], ssems[b],
                             add=True)

        def sc_wait(b, cd):
            pltpu.make_async_copy(rows[b], accum.at[dstv2d.at[cd]],
                                  ssems[b]).wait()

        def sk_body(j, carry):
            base = s * EPER + j * SK
            pltpu.async_copy(src_hbm.at[pl.ds(base, SK)], srcv, isem)
            pltpu.async_copy(dst4d_hbm.at[s, j], dstv2d, isem)
            pltpu.make_async_copy(src_hbm.at[pl.ds(base, SK)], srcv,
                                  isem).wait()
            pltpu.make_async_copy(dst4d_hbm.at[s, j], dstv2d, isem).wait()

            def adj(g, carry2):
                srcv[pl.ds(g * 16, 16)] = srcv[pl.ds(g * 16, 16)] + cN
                return carry2

            lax.fori_loop(0, SK // 16, adj, 0)
            g_start(0, 0, base)
            g_start(1, 1, base)

            def quad(p, carry2):
                c0 = 4 * p
                for t in range(4):
                    cd = c0 + t
                    b = t
                    b2 = (t + 2) % 4
                    if t >= 2:
                        sc_wait(b2, cd - 2)
                    else:
                        @pl.when(p > 0)
                        def _():
                            sc_wait(b2, cd - 2)
                    if t < 3:
                        g_start(b2, cd + 2, base)
                    else:
                        @pl.when(p < NQ - 1)
                        def _():
                            g_start(b2, cd + 2, base)
                    g_wait(b, cd, base)
                    compute(b, cd)
                    sc_start(b, cd)
                return carry2

            lax.fori_loop(0, NQ, quad, 0)
            sc_wait(2, NC2 - 3)
            g_wait(0, NC2 - 1, base)
            compute(0, NC2 - 1)
            sc_start(0, NC2 - 1)
            sc_wait(3, NC2 - 2)
            sc_wait(0, NC2 - 1)
            return carry

        lax.fori_loop(0, NSK, sk_body, 0)
        plsc.subcore_barrier()
        pltpu.sync_copy(accum.at[pl.ds(rbase, NPT)],
                        agg_hbm.at[pl.ds(cN + rbase, NPT)])

        @pl.when(s == 0)
        def _():
            pltpu.sync_copy(accum.at[pl.ds(NSUB * NPT, NTAIL)],
                            agg_hbm.at[pl.ds(cN + NSUB * NPT, NTAIL)])

    return k(fcS, a_hbm_in, src, dst4d, zeros_hbm)


def _final_body(a0_ref, a1_ref, fdst_ref, wa_ref, ba_ref, sc_ref, of_ref,
                out_ref):
    x = jnp.concatenate([a0_ref[...], a1_ref[...]], axis=1)
    r = lax.broadcasted_iota(jnp.int32, (HF, HF), 0) // F
    cc = lax.broadcasted_iota(jnp.int32, (HF, HF), 1) // F
    M = jnp.where(r == cc, jnp.float32(1.0 / F), jnp.float32(0.0))
    mean = jnp.dot(x, M, preferred_element_type=jnp.float32)
    xc = x - mean
    var = jnp.dot(xc * xc, M, preferred_element_type=jnp.float32) + 1e-9
    y = xc * sc_ref[...] * lax.rsqrt(var) + of_ref[...]
    out_ref[...] = _matT(y, wa_ref[...]) + ba_ref[...] + fdst_ref[...]


def _final(agg, fdst, W_agg, b_agg, scale2, offset2):
    R = 1000
    GN = N // R
    return pl.pallas_call(
        _final_body,
        grid=(GN,),
        in_specs=[
            pl.BlockSpec((R, HHALF), lambda i: (i, 0)),
            pl.BlockSpec((R, HHALF), lambda i: (GN + i, 0)),
            pl.BlockSpec((R, HF), lambda i: (i, 0)),
            pl.BlockSpec((HF, HF), lambda i: (0, 0)),
            pl.BlockSpec((1, HF), lambda i: (0, 0)),
            pl.BlockSpec((1, HF), lambda i: (0, 0)),
            pl.BlockSpec((1, HF), lambda i: (0, 0)),
        ],
        out_specs=pl.BlockSpec((R, HF), lambda i: (i, 0)),
        out_shape=jax.ShapeDtypeStruct((N, HF), jnp.float32),
    )(agg, agg, fdst, W_agg, b_agg, scale2, offset2)


def kernel(feat_src, edge_index, feat_edge, W_src, W_attn_src, W_attn_dst,
           W_attn_edge, scale, offset, W_agg, b_agg, W_dst, b_dst):
    W_cat = jnp.concatenate([W_attn_src, W_attn_dst], axis=0)
    fcS, attn_tbl, fdst, aeT = _node_proj(
        feat_src, W_src, W_cat, W_dst, b_dst.reshape(1, HF),
        feat_edge, W_attn_edge)
    src = edge_index[0]
    dst = edge_index[1]
    a_edge = _sc_attn_kernel(attn_tbl.reshape(N * 2 * H),
                             aeT.reshape(H * E), src, dst)
    zeros = jnp.zeros((NPT, HHALF), jnp.float32)
    dst4d = dst.reshape(NSUB, NSK, NC2, K)
    agg = _sc_edge_kernel(fcS, a_edge, src, dst4d, zeros)
    rst = _final(agg, fdst, W_agg, b_agg.reshape(1, HF),
                 scale.reshape(1, HF), offset.reshape(1, HF))
    return rst.reshape(N, H, F)
